# baseline (speedup 1.0000x reference)
import jax
import jax.numpy as jnp
from jax.experimental import pallas as pl
from jax.experimental.pallas import tpu as pltpu

_CHUNK = 1048576
_SUB = 1024


def _bce_body(x_ref, t_ref, out_ref, acc_ref):
    i = pl.program_id(0)

    @pl.when(i == 0)
    def _init():
        acc_ref[...] = jnp.zeros_like(acc_ref)

    def step(j, carry):
        s, c = carry
        x = x_ref[pl.ds(j * _SUB, _SUB)].reshape(8, 128)
        t = t_ref[pl.ds(j * _SUB, _SUB)].reshape(8, 128)
        tf = t.astype(jnp.float32)
        zf = jnp.maximum(tf, 0.0)
        mf = jnp.minimum(tf + 1.0, 1.0)
        sp = jnp.maximum(x, 0.0) + jnp.log1p(jnp.exp(-jnp.abs(x)))
        return s + (mf * sp - x * zf), c + mf

    init = (jnp.zeros((8, 128), jnp.float32),
            jnp.zeros((8, 128), jnp.float32))
    s, c = jax.lax.fori_loop(0, _CHUNK // _SUB, step, init, unroll=8)
    acc_ref[0] += s
    acc_ref[1] += c

    @pl.when(i == pl.num_programs(0) - 1)
    def _fin():
        out_ref[0] = jnp.sum(acc_ref[0]) / jnp.sum(acc_ref[1])


def kernel(output, target):
    n = output.shape[0]
    grid = n // _CHUNK

    out = pl.pallas_call(
        _bce_body,
        grid=(grid,),
        in_specs=[
            pl.BlockSpec((_CHUNK,), lambda i: (i,)),
            pl.BlockSpec((_CHUNK,), lambda i: (i,)),
        ],
        out_specs=pl.BlockSpec(memory_space=pltpu.SMEM),
        out_shape=jax.ShapeDtypeStruct((1,), jnp.float32),
        scratch_shapes=[pltpu.VMEM((2, 8, 128), jnp.float32)],
    )(output, target)
    return out[0]


# speedup vs baseline: 1.2182x; 1.2182x over previous
import jax
import jax.numpy as jnp
from jax.experimental import pallas as pl
from jax.experimental.pallas import tpu as pltpu

_CHUNK = 1048576
_SUB = 8192
_ROWS = _SUB // 128


def _bce_body(x_ref, t_ref, out_ref, acc_ref):
    i = pl.program_id(0)

    @pl.when(i == 0)
    def _init():
        acc_ref[...] = jnp.zeros_like(acc_ref)

    def step(j, carry):
        s, c = carry
        x = x_ref[pl.ds(j * _SUB, _SUB)].reshape(_ROWS, 128)
        t = t_ref[pl.ds(j * _SUB, _SUB)].reshape(_ROWS, 128)
        tf = t.astype(jnp.float32)
        zf = jnp.maximum(tf, 0.0)
        mf = jnp.minimum(tf + 1.0, 1.0)
        sp = jnp.maximum(x, 0.0) + jnp.log1p(jnp.exp(-jnp.abs(x)))
        return s + (mf * sp - x * zf), c + mf

    init = (jnp.zeros((_ROWS, 128), jnp.float32),
            jnp.zeros((_ROWS, 128), jnp.float32))
    s, c = jax.lax.fori_loop(0, _CHUNK // _SUB, step, init, unroll=2)
    acc_ref[0] += s
    acc_ref[1] += c

    @pl.when(i == pl.num_programs(0) - 1)
    def _fin():
        out_ref[0] = jnp.sum(acc_ref[0]) / jnp.sum(acc_ref[1])


def kernel(output, target):
    n = output.shape[0]
    grid = n // _CHUNK

    out = pl.pallas_call(
        _bce_body,
        grid=(grid,),
        in_specs=[
            pl.BlockSpec((_CHUNK,), lambda i: (i,)),
            pl.BlockSpec((_CHUNK,), lambda i: (i,)),
        ],
        out_specs=pl.BlockSpec(memory_space=pltpu.SMEM),
        out_shape=jax.ShapeDtypeStruct((1,), jnp.float32),
        scratch_shapes=[pltpu.VMEM((2, _ROWS, 128), jnp.float32)],
    )(output, target)
    return out[0]


# speedup vs baseline: 1.2365x; 1.0150x over previous
import jax
import jax.numpy as jnp
from jax.experimental import pallas as pl
from jax.experimental.pallas import tpu as pltpu

_CHUNK = 1048576
_SUB = 16384
_ROWS = _SUB // 128


def _bce_body(x_ref, t_ref, out_ref, acc_ref):
    i = pl.program_id(0)

    @pl.when(i == 0)
    def _init():
        acc_ref[...] = jnp.zeros_like(acc_ref)

    def step(j, carry):
        s, c = carry
        x = x_ref[pl.ds(j * _SUB, _SUB)].reshape(_ROWS, 128)
        t = t_ref[pl.ds(j * _SUB, _SUB)].reshape(_ROWS, 128)
        tf = t.astype(jnp.float32)
        zf = jnp.maximum(tf, 0.0)
        mf = jnp.minimum(tf + 1.0, 1.0)
        sp = jnp.maximum(x, 0.0) + jnp.log1p(jnp.exp(-jnp.abs(x)))
        return s + (mf * sp - x * zf), c + mf

    init = (jnp.zeros((_ROWS, 128), jnp.float32),
            jnp.zeros((_ROWS, 128), jnp.float32))
    s, c = jax.lax.fori_loop(0, _CHUNK // _SUB, step, init, unroll=2)
    acc_ref[0] += s
    acc_ref[1] += c

    @pl.when(i == pl.num_programs(0) - 1)
    def _fin():
        out_ref[0] = jnp.sum(acc_ref[0]) / jnp.sum(acc_ref[1])


def kernel(output, target):
    n = output.shape[0]
    grid = n // _CHUNK

    out = pl.pallas_call(
        _bce_body,
        grid=(grid,),
        in_specs=[
            pl.BlockSpec((_CHUNK,), lambda i: (i,)),
            pl.BlockSpec((_CHUNK,), lambda i: (i,)),
        ],
        out_specs=pl.BlockSpec(memory_space=pltpu.SMEM),
        out_shape=jax.ShapeDtypeStruct((1,), jnp.float32),
        scratch_shapes=[pltpu.VMEM((2, _ROWS, 128), jnp.float32)],
    )(output, target)
    return out[0]
